# Initial kernel scaffold; baseline (speedup 1.0000x reference)
import jax
import jax.numpy as jnp
from jax.experimental import pallas as pl


def kernel(input_tok, input_tok_type, input_tok_pos, input_ent, input_ent_type, ent_candidates, input_ent_pos, word_emb, ent_emb, pos_emb, ent_row_pos_emb, ent_col_pos_emb, type_emb, ln_gamma, ln_beta):
    raise NotImplementedError("write your pallas kernel here")


# speedup vs baseline: 2.5375x; 2.5375x over previous
import functools

import jax
import jax.numpy as jnp
from jax import lax
from jax.experimental import pallas as pl
from jax.experimental.pallas import tpu as pltpu
from jax.experimental.pallas import tpu_sc as plsc

B = 1024
L = 50
LE = 50
C = 256
H = 128
EPS = 1e-12

NC = 2
NS = 16
NW = NC * NS

TOK_N = B * L
ENT_N = B * LE
CAND_N = B * C

TOK_PER_W = TOK_N // NW
ENT_PER_W = ENT_N // NW
CAND_PER_W = CAND_N // NW

KB = 80
KC = 128

_NLN = H // 16


def _rsqrt_vec(v):
    i = plsc.bitcast(v, jnp.int32)
    i = jnp.int32(0x5F3759DF) - (i >> 1)
    y = plsc.bitcast(i, jnp.float32)
    for _ in range(3):
        y = y * (1.5 - 0.5 * v * y * y)
    return y


def _gather_chunk(tables, idx_hbms, idx_bufs, row_bufs, base, k, sem):
    for idx_hbm, idx_buf in zip(idx_hbms, idx_bufs):
        pltpu.sync_copy(idx_hbm.at[pl.ds(base, k)], idx_buf)
    handles = []
    for table, idx_buf, row_buf in zip(tables, idx_bufs, row_bufs):
        handles.append(pltpu.async_copy(table.at[idx_buf], row_buf, sem))
    for h in handles:
        h.wait()


def _ln_rows(row_bufs, out_buf, gamma_v, beta_v, k):
    g = [gamma_v[pl.ds(16 * j, 16)] for j in range(_NLN)]
    bta = [beta_v[pl.ds(16 * j, 16)] for j in range(_NLN)]

    def body(r, _):
        xs = []
        for j in range(_NLN):
            x = row_bufs[0][r, pl.ds(16 * j, 16)]
            for rb in row_bufs[1:]:
                x = x + rb[r, pl.ds(16 * j, 16)]
            xs.append(x)
        s = xs[0]
        for x in xs[1:]:
            s = s + x
        m = jnp.sum(s) * (1.0 / H)
        m_vec = lax.broadcast(m, (16,))
        ds = [x - m_vec for x in xs]
        s2 = ds[0] * ds[0]
        for dj in ds[1:]:
            s2 = s2 + dj * dj
        var = jnp.sum(s2) * (1.0 / H)
        r_vec = _rsqrt_vec(lax.broadcast(var + EPS, (16,)))
        for j in range(_NLN):
            out_buf[r, pl.ds(16 * j, 16)] = ds[j] * (r_vec * g[j]) + bta[j]
        return 0

    lax.fori_loop(0, k, body, 0)


def _sc_body(tok_i, tokpos_i, toktype_i, ent_i, enttype_i, entrow_i,
             entcol_i, cand_i, word_emb, ent_emb, pos_emb, row_emb,
             col_emb, type_emb, gamma, beta,
             out_tok, out_ent, out_cand,
             i0, i1, i2, i3, b0, b1, b2, b3, ci, cb, gamma_v, beta_v, sem):
    wid = lax.axis_index("s") * NC + lax.axis_index("c")

    pltpu.sync_copy(gamma, gamma_v)
    pltpu.sync_copy(beta, beta_v)

    def tok_chunk(c, _):
        base = wid * TOK_PER_W + c * KB
        _gather_chunk((word_emb, pos_emb, type_emb),
                      (tok_i, tokpos_i, toktype_i),
                      (i0, i1, i2), (b0, b1, b2), base, KB, sem)
        _ln_rows((b0, b1, b2), b0, gamma_v, beta_v, KB)
        pltpu.sync_copy(b0, out_tok.at[pl.ds(base, KB)])
        return 0

    lax.fori_loop(0, TOK_PER_W // KB, tok_chunk, 0)

    def ent_chunk(c, _):
        base = wid * ENT_PER_W + c * KB
        _gather_chunk((ent_emb, type_emb, row_emb, col_emb),
                      (ent_i, enttype_i, entrow_i, entcol_i),
                      (i0, i1, i2, i3), (b0, b1, b2, b3), base, KB, sem)
        _ln_rows((b0, b1, b2, b3), b0, gamma_v, beta_v, KB)
        pltpu.sync_copy(b0, out_ent.at[pl.ds(base, KB)])
        return 0

    lax.fori_loop(0, ENT_PER_W // KB, ent_chunk, 0)

    def cand_chunk(c, _):
        base = wid * CAND_PER_W + c * KC
        pltpu.sync_copy(cand_i.at[pl.ds(base, KC)], ci)
        pltpu.async_copy(ent_emb.at[ci], cb, sem).wait()
        pltpu.sync_copy(cb, out_cand.at[pl.ds(base, KC)])
        return 0

    lax.fori_loop(0, CAND_PER_W // KC, cand_chunk, 0)


@jax.jit
def kernel(input_tok, input_tok_type, input_tok_pos, input_ent,
           input_ent_type, ent_candidates, input_ent_pos, word_emb, ent_emb,
           pos_emb, ent_row_pos_emb, ent_col_pos_emb, type_emb, ln_gamma,
           ln_beta):
    mesh = plsc.VectorSubcoreMesh(core_axis_name="c", subcore_axis_name="s",
                                  num_cores=NC, num_subcores=NS)
    f = pl.kernel(
        _sc_body,
        out_type=(
            jax.ShapeDtypeStruct((TOK_N, H), jnp.float32),
            jax.ShapeDtypeStruct((ENT_N, H), jnp.float32),
            jax.ShapeDtypeStruct((CAND_N, H), jnp.float32),
        ),
        mesh=mesh,
        compiler_params=pltpu.CompilerParams(needs_layout_passes=False),
        scratch_types=[
            pltpu.VMEM((KB,), jnp.int32),
            pltpu.VMEM((KB,), jnp.int32),
            pltpu.VMEM((KB,), jnp.int32),
            pltpu.VMEM((KB,), jnp.int32),
            pltpu.VMEM((KB, H), jnp.float32),
            pltpu.VMEM((KB, H), jnp.float32),
            pltpu.VMEM((KB, H), jnp.float32),
            pltpu.VMEM((KB, H), jnp.float32),
            pltpu.VMEM((KC,), jnp.int32),
            pltpu.VMEM((KC, H), jnp.float32),
            pltpu.VMEM((H,), jnp.float32),
            pltpu.VMEM((H,), jnp.float32),
            pltpu.SemaphoreType.DMA,
        ],
    )
    out_tok, out_ent, out_cand = f(
        input_tok.reshape(-1), input_tok_pos.reshape(-1),
        input_tok_type.reshape(-1), input_ent.reshape(-1),
        input_ent_type.reshape(-1),
        input_ent_pos[..., 0].reshape(-1).copy(),
        input_ent_pos[..., 1].reshape(-1).copy(),
        ent_candidates.reshape(-1),
        word_emb, ent_emb, pos_emb, ent_row_pos_emb, ent_col_pos_emb,
        type_emb, ln_gamma, ln_beta)
    return (out_tok.reshape(B, L, H), out_ent.reshape(B, LE, H),
            out_cand.reshape(B, C, H))


# speedup vs baseline: 2.8139x; 1.1089x over previous
import functools

import jax
import jax.numpy as jnp
from jax import lax
from jax.experimental import pallas as pl
from jax.experimental.pallas import tpu as pltpu
from jax.experimental.pallas import tpu_sc as plsc

B = 1024
L = 50
LE = 50
C = 256
H = 128
EPS = 1e-12

NC = 2
NS = 16
NW = NC * NS

TOK_N = B * L
ENT_N = B * LE
CAND_N = B * C

TOK_PER_W = TOK_N // NW
ENT_PER_W = ENT_N // NW
CAND_PER_W = CAND_N // NW

KB = 80
KC = 128
NCH_B = TOK_PER_W // KB
NCH_C = CAND_PER_W // KC

_NLN = H // 16


def _rsqrt_vec(v):
    i = plsc.bitcast(v, jnp.int32)
    i = jnp.int32(0x5F3759DF) - (i >> 1)
    y = plsc.bitcast(i, jnp.float32)
    for _ in range(3):
        y = y * (1.5 - 0.5 * v * y * y)
    return y


def _ln_rows(row_bufs, out_buf, gamma_v, beta_v, k):
    g = [gamma_v[pl.ds(16 * j, 16)] for j in range(_NLN)]
    bta = [beta_v[pl.ds(16 * j, 16)] for j in range(_NLN)]

    def body(r, _):
        xs = []
        for j in range(_NLN):
            x = row_bufs[0][r, pl.ds(16 * j, 16)]
            for rb in row_bufs[1:]:
                x = x + rb[r, pl.ds(16 * j, 16)]
            xs.append(x)
        s = xs[0]
        for x in xs[1:]:
            s = s + x
        m = jnp.sum(s) * (1.0 / H)
        m_vec = lax.broadcast(m, (16,))
        ds = [x - m_vec for x in xs]
        s2 = ds[0] * ds[0]
        for dj in ds[1:]:
            s2 = s2 + dj * dj
        var = jnp.sum(s2) * (1.0 / H)
        r_vec = _rsqrt_vec(lax.broadcast(var + EPS, (16,)))
        for j in range(_NLN):
            out_buf[r, pl.ds(16 * j, 16)] = ds[j] * (r_vec * g[j]) + bta[j]
        return 0

    lax.fori_loop(0, k, body, 0)


def _sc_body(tok_i, tokpos_i, toktype_i, ent_i, enttype_i, entrow_i,
             entcol_i, cand_i, word_emb, ent_emb, pos_emb, row_emb,
             col_emb, type_emb, gamma, beta,
             out_tok, out_ent, out_cand,
             i0, i1, i2, i3, i4, i5, i6, i7,
             b0, b1, b2, b3, b4, b5, b6, b7,
             ci2d, cb0, cb1, gamma_v, beta_v, sg0, sg1):
    wid = lax.axis_index("s") * NC + lax.axis_index("c")

    pltpu.sync_copy(gamma, gamma_v)
    pltpu.sync_copy(beta, beta_v)

    idx_slots = ((i0, i1, i2, i3), (i4, i5, i6, i7))
    buf_slots = ((b0, b1, b2, b3), (b4, b5, b6, b7))
    sems = (sg0, sg1)

    def run_ln_branch(tables, idx_hbms, n_per_w, out_hbm):
        nt = len(tables)

        def fire(c, b):
            base = wid * n_per_w + c * KB
            for idx_hbm, ib in zip(idx_hbms, idx_slots[b][:nt]):
                pltpu.sync_copy(idx_hbm.at[pl.ds(base, KB)], ib)
            for table, ib, rb in zip(tables, idx_slots[b][:nt],
                                     buf_slots[b][:nt]):
                pltpu.async_copy(table.at[ib], rb, sems[b])

        fire(0, 0)
        fire(1, 1)

        def outer(gidx, _):
            for b in range(2):
                c = 2 * gidx + b
                for table, ib, rb in zip(tables, idx_slots[b][:nt],
                                         buf_slots[b][:nt]):
                    pltpu.make_async_copy(table.at[ib], rb, sems[b]).wait()
                _ln_rows(buf_slots[b][:nt], buf_slots[b][0], gamma_v,
                         beta_v, KB)
                base = wid * n_per_w + c * KB
                pltpu.sync_copy(buf_slots[b][0], out_hbm.at[pl.ds(base, KB)])

                @pl.when(c + 2 < NCH_B)
                def _():
                    fire(c + 2, b)
            return 0

        lax.fori_loop(0, NCH_B // 2, outer, 0)

    run_ln_branch((word_emb, pos_emb, type_emb),
                  (tok_i, tokpos_i, toktype_i), TOK_PER_W, out_tok)
    run_ln_branch((ent_emb, type_emb, row_emb, col_emb),
                  (ent_i, enttype_i, entrow_i, entcol_i), ENT_PER_W, out_ent)

    pltpu.sync_copy(cand_i.at[wid], ci2d)
    cbufs = (cb0, cb1)

    def cfire(c, b):
        pltpu.async_copy(ent_emb.at[ci2d.at[c]], cbufs[b], sems[b])

    cfire(0, 0)
    cfire(1, 1)

    def couter(gidx, _):
        for b in range(2):
            c = 2 * gidx + b
            pltpu.make_async_copy(ent_emb.at[ci2d.at[c]], cbufs[b],
                                  sems[b]).wait()
            base = wid * CAND_PER_W + c * KC
            pltpu.sync_copy(cbufs[b], out_cand.at[pl.ds(base, KC)])

            @pl.when(c + 2 < NCH_C)
            def _():
                cfire(c + 2, b)
        return 0

    lax.fori_loop(0, NCH_C // 2, couter, 0)


@jax.jit
def kernel(input_tok, input_tok_type, input_tok_pos, input_ent,
           input_ent_type, ent_candidates, input_ent_pos, word_emb, ent_emb,
           pos_emb, ent_row_pos_emb, ent_col_pos_emb, type_emb, ln_gamma,
           ln_beta):
    mesh = plsc.VectorSubcoreMesh(core_axis_name="c", subcore_axis_name="s",
                                  num_cores=NC, num_subcores=NS)
    f = pl.kernel(
        _sc_body,
        out_type=(
            jax.ShapeDtypeStruct((TOK_N, H), jnp.float32),
            jax.ShapeDtypeStruct((ENT_N, H), jnp.float32),
            jax.ShapeDtypeStruct((CAND_N, H), jnp.float32),
        ),
        mesh=mesh,
        compiler_params=pltpu.CompilerParams(needs_layout_passes=False),
        scratch_types=[
            pltpu.VMEM((KB,), jnp.int32), pltpu.VMEM((KB,), jnp.int32),
            pltpu.VMEM((KB,), jnp.int32), pltpu.VMEM((KB,), jnp.int32),
            pltpu.VMEM((KB,), jnp.int32), pltpu.VMEM((KB,), jnp.int32),
            pltpu.VMEM((KB,), jnp.int32), pltpu.VMEM((KB,), jnp.int32),
            pltpu.VMEM((KB, H), jnp.float32), pltpu.VMEM((KB, H), jnp.float32),
            pltpu.VMEM((KB, H), jnp.float32), pltpu.VMEM((KB, H), jnp.float32),
            pltpu.VMEM((KB, H), jnp.float32), pltpu.VMEM((KB, H), jnp.float32),
            pltpu.VMEM((KB, H), jnp.float32), pltpu.VMEM((KB, H), jnp.float32),
            pltpu.VMEM((NCH_C, KC), jnp.int32),
            pltpu.VMEM((KC, H), jnp.float32), pltpu.VMEM((KC, H), jnp.float32),
            pltpu.VMEM((H,), jnp.float32), pltpu.VMEM((H,), jnp.float32),
            pltpu.SemaphoreType.DMA, pltpu.SemaphoreType.DMA,
        ],
    )
    out_tok, out_ent, out_cand = f(
        input_tok.reshape(-1), input_tok_pos.reshape(-1),
        input_tok_type.reshape(-1), input_ent.reshape(-1),
        input_ent_type.reshape(-1),
        input_ent_pos[..., 0].reshape(-1).copy(),
        input_ent_pos[..., 1].reshape(-1).copy(),
        ent_candidates.reshape(NW, NCH_C, KC),
        word_emb, ent_emb, pos_emb, ent_row_pos_emb, ent_col_pos_emb,
        type_emb, ln_gamma, ln_beta)
    return (out_tok.reshape(B, L, H), out_ent.reshape(B, LE, H),
            out_cand.reshape(B, C, H))


# speedup vs baseline: 2.8452x; 1.0111x over previous
import functools

import jax
import jax.numpy as jnp
from jax import lax
from jax.experimental import pallas as pl
from jax.experimental.pallas import tpu as pltpu
from jax.experimental.pallas import tpu_sc as plsc

B = 1024
L = 50
LE = 50
C = 256
H = 128
EPS = 1e-12

NC = 2
NS = 16
NW = NC * NS

TOK_N = B * L
ENT_N = B * LE
CAND_N = B * C

TOK_PER_W = TOK_N // NW
ENT_PER_W = ENT_N // NW
CAND_PER_W = CAND_N // NW

KB = 80
KC = 128
NCH_B = TOK_PER_W // KB
NCH_C = CAND_PER_W // KC

_NLN = H // 16


def _rsqrt_vec(v):
    i = plsc.bitcast(v, jnp.int32)
    i = jnp.int32(0x5F3759DF) - (i >> 1)
    y = plsc.bitcast(i, jnp.float32)
    for _ in range(3):
        y = y * (1.5 - 0.5 * v * y * y)
    return y


def _ln_rows(row_bufs, out_buf, gamma_v, beta_v, k):
    g = [gamma_v[pl.ds(16 * j, 16)] for j in range(_NLN)]
    bta = [beta_v[pl.ds(16 * j, 16)] for j in range(_NLN)]

    def body(r, _):
        xs = []
        for j in range(_NLN):
            x = row_bufs[0][r, pl.ds(16 * j, 16)]
            for rb in row_bufs[1:]:
                x = x + rb[r, pl.ds(16 * j, 16)]
            xs.append(x)
        s = xs[0]
        for x in xs[1:]:
            s = s + x
        m = jnp.sum(s) * (1.0 / H)
        m_vec = lax.broadcast(m, (16,))
        ds = [x - m_vec for x in xs]
        s2 = ds[0] * ds[0]
        for dj in ds[1:]:
            s2 = s2 + dj * dj
        var = jnp.sum(s2) * (1.0 / H)
        r_vec = _rsqrt_vec(lax.broadcast(var + EPS, (16,)))
        for j in range(_NLN):
            out_buf[r, pl.ds(16 * j, 16)] = ds[j] * (r_vec * g[j]) + bta[j]
        return 0

    lax.fori_loop(0, k, body, 0)


def _sc_body(tok_i, tokpos_i, toktype_i, ent_i, enttype_i, entrow_i,
             entcol_i, cand_i, word_emb, ent_emb, pos_emb, row_emb,
             col_emb, type_emb, gamma, beta,
             out_tok, out_ent, out_cand,
             i0, i1, i2, i3, i4, i5, i6, i7,
             b0, b1, b2, b3, b4, b5, b6, b7,
             ci2d, cb0, cb1, gamma_v, beta_v, sg0, sg1):
    wid = lax.axis_index("s") * NC + lax.axis_index("c")

    pltpu.sync_copy(gamma, gamma_v)
    pltpu.sync_copy(beta, beta_v)

    idx_slots = ((i0, i1, i2, i3), (i4, i5, i6, i7))
    buf_slots = ((b0, b1, b2, b3), (b4, b5, b6, b7))
    sems = (sg0, sg1)

    def run_ln_branch(tables, idx_hbms, n_per_w, out_hbm):
        nt = len(tables)

        def fire(c, b):
            base = wid * n_per_w + c * KB
            for idx_hbm, ib in zip(idx_hbms, idx_slots[b][:nt]):
                pltpu.sync_copy(idx_hbm.at[pl.ds(base, KB)], ib)
            for table, ib, rb in zip(tables, idx_slots[b][:nt],
                                     buf_slots[b][:nt]):
                pltpu.async_copy(table.at[ib], rb, sems[b])

        fire(0, 0)
        fire(1, 1)

        def outer(gidx, _):
            for b in range(2):
                c = 2 * gidx + b
                for table, ib, rb in zip(tables, idx_slots[b][:nt],
                                         buf_slots[b][:nt]):
                    pltpu.make_async_copy(table.at[ib], rb, sems[b]).wait()
                _ln_rows(buf_slots[b][:nt], buf_slots[b][0], gamma_v,
                         beta_v, KB)
                base = wid * n_per_w + c * KB
                pltpu.sync_copy(buf_slots[b][0], out_hbm.at[pl.ds(base, KB)])

                @pl.when(c + 2 < NCH_B)
                def _():
                    fire(c + 2, b)
            return 0

        lax.fori_loop(0, NCH_B // 2, outer, 0)

    with jax.named_scope("tok_branch"):
        run_ln_branch((word_emb, pos_emb, type_emb),
                      (tok_i, tokpos_i, toktype_i), TOK_PER_W, out_tok)
    with jax.named_scope("ent_branch"):
        run_ln_branch((ent_emb, type_emb, row_emb, col_emb),
                      (ent_i, enttype_i, entrow_i, entcol_i), ENT_PER_W,
                      out_ent)

    pltpu.sync_copy(cand_i.at[wid], ci2d)
    cbufs = (cb0, cb1)

    def cfire(c, b):
        pltpu.async_copy(ent_emb.at[ci2d.at[c]], cbufs[b], sems[b])

    cfire(0, 0)
    cfire(1, 1)

    def couter(gidx, _):
        for b in range(2):
            c = 2 * gidx + b
            pltpu.make_async_copy(ent_emb.at[ci2d.at[c]], cbufs[b],
                                  sems[b]).wait()
            base = wid * CAND_PER_W + c * KC
            pltpu.sync_copy(cbufs[b], out_cand.at[pl.ds(base, KC)])

            @pl.when(c + 2 < NCH_C)
            def _():
                cfire(c + 2, b)
        return 0

    with jax.named_scope("cand_branch"):
        lax.fori_loop(0, NCH_C // 2, couter, 0)


@jax.jit
def kernel(input_tok, input_tok_type, input_tok_pos, input_ent,
           input_ent_type, ent_candidates, input_ent_pos, word_emb, ent_emb,
           pos_emb, ent_row_pos_emb, ent_col_pos_emb, type_emb, ln_gamma,
           ln_beta):
    mesh = plsc.VectorSubcoreMesh(core_axis_name="c", subcore_axis_name="s",
                                  num_cores=NC, num_subcores=NS)
    f = pl.kernel(
        _sc_body,
        out_type=(
            jax.ShapeDtypeStruct((TOK_N, H), jnp.float32),
            jax.ShapeDtypeStruct((ENT_N, H), jnp.float32),
            jax.ShapeDtypeStruct((CAND_N, H), jnp.float32),
        ),
        mesh=mesh,
        compiler_params=pltpu.CompilerParams(needs_layout_passes=False),
        scratch_types=[
            pltpu.VMEM((KB,), jnp.int32), pltpu.VMEM((KB,), jnp.int32),
            pltpu.VMEM((KB,), jnp.int32), pltpu.VMEM((KB,), jnp.int32),
            pltpu.VMEM((KB,), jnp.int32), pltpu.VMEM((KB,), jnp.int32),
            pltpu.VMEM((KB,), jnp.int32), pltpu.VMEM((KB,), jnp.int32),
            pltpu.VMEM((KB, H), jnp.float32), pltpu.VMEM((KB, H), jnp.float32),
            pltpu.VMEM((KB, H), jnp.float32), pltpu.VMEM((KB, H), jnp.float32),
            pltpu.VMEM((KB, H), jnp.float32), pltpu.VMEM((KB, H), jnp.float32),
            pltpu.VMEM((KB, H), jnp.float32), pltpu.VMEM((KB, H), jnp.float32),
            pltpu.VMEM((NCH_C, KC), jnp.int32),
            pltpu.VMEM((KC, H), jnp.float32), pltpu.VMEM((KC, H), jnp.float32),
            pltpu.VMEM((H,), jnp.float32), pltpu.VMEM((H,), jnp.float32),
            pltpu.SemaphoreType.DMA, pltpu.SemaphoreType.DMA,
        ],
    )
    out_tok, out_ent, out_cand = f(
        input_tok.reshape(-1), input_tok_pos.reshape(-1),
        input_tok_type.reshape(-1), input_ent.reshape(-1),
        input_ent_type.reshape(-1),
        input_ent_pos[..., 0].reshape(-1).copy(),
        input_ent_pos[..., 1].reshape(-1).copy(),
        ent_candidates.reshape(NW, NCH_C, KC),
        word_emb, ent_emb, pos_emb, ent_row_pos_emb, ent_col_pos_emb,
        type_emb, ln_gamma, ln_beta)
    return (out_tok.reshape(B, L, H), out_ent.reshape(B, LE, H),
            out_cand.reshape(B, C, H))
